# baseline (speedup 1.0000x reference)
import jax
import jax.numpy as jnp
from jax import lax
from jax.experimental import pallas as pl
from jax.experimental.pallas import tpu as pltpu

_ALPHA = 0.4
_R = 8
_NBUF = 4
_INTERPRET = False


def _mix_body(perm_ref, ap_ref, aq_ref, lam_ref, hp_ref, hq_ref,
              op_ref, oq_ref, bufs_p, bufs_q, semp, semq):
    i = pl.program_id(0)
    n = pl.num_programs(0)
    rem = lax.rem(i, _NBUF)

    def issue(step, s):
        base = step * _R
        for j in range(_R):
            row = perm_ref[base + j]
            pltpu.make_async_copy(
                hp_ref.at[pl.ds(row, 1)], bufs_p[s].at[pl.ds(j, 1)],
                semp.at[s]).start()
            pltpu.make_async_copy(
                hq_ref.at[pl.ds(row, 1)], bufs_q[s].at[pl.ds(j, 1)],
                semq.at[s]).start()

    @pl.when(i == 0)
    def _():
        for k in range(_NBUF - 1):
            issue(k, k)

    lam = lam_ref[...]
    one_m = 1.0 - lam

    def step_for(s):
        s2 = (s + _NBUF - 1) % _NBUF

        @pl.when(i + _NBUF - 1 < n)
        def _():
            issue(i + _NBUF - 1, s2)

        for j in range(_R):
            pltpu.make_async_copy(
                hp_ref.at[pl.ds(0, 1)], bufs_p[s].at[pl.ds(j, 1)],
                semp.at[s]).wait()
            pltpu.make_async_copy(
                hq_ref.at[pl.ds(0, 1)], bufs_q[s].at[pl.ds(j, 1)],
                semq.at[s]).wait()
        a = ap_ref[...]
        b = bufs_p[s][...]
        op_ref[...] = a + jnp.log(lam + one_m * jnp.exp(b - a))
        a = aq_ref[...]
        b = bufs_q[s][...]
        oq_ref[...] = a + jnp.log(lam + one_m * jnp.exp(b - a))

    for s in range(_NBUF):
        @pl.when(rem == s)
        def _(s=s):
            step_for(s)


def kernel(x_pre, x_post):
    b, g = x_pre.shape
    key = jax.random.key(1)
    kp, kl = jax.random.split(key)
    perm = jax.random.permutation(kp, b)
    lam = jax.random.beta(kl, _ALPHA, _ALPHA, (b,)).astype(jnp.float32)

    hbm = pl.BlockSpec(memory_space=pl.ANY)
    lam_spec = pl.BlockSpec((_R, 1), lambda i, perm_r: (i, 0))
    out_spec = pl.BlockSpec((_R, g), lambda i, perm_r: (i, 0))

    grid_spec = pltpu.PrefetchScalarGridSpec(
        num_scalar_prefetch=1,
        grid=(b // _R,),
        in_specs=[out_spec, out_spec, lam_spec, hbm, hbm],
        out_specs=[out_spec, out_spec],
        scratch_shapes=[
            [pltpu.VMEM((_R, g), jnp.float32) for _ in range(_NBUF)],
            [pltpu.VMEM((_R, g), jnp.float32) for _ in range(_NBUF)],
            pltpu.SemaphoreType.DMA((_NBUF,)),
            pltpu.SemaphoreType.DMA((_NBUF,)),
        ],
    )
    out_shape = [jax.ShapeDtypeStruct((b, g), jnp.float32)] * 2
    op, oq = pl.pallas_call(
        _mix_body,
        grid_spec=grid_spec,
        out_shape=out_shape,
        interpret=_INTERPRET,
    )(perm, x_pre, x_post, lam.reshape(b, 1), x_pre, x_post)
    return op, oq, lam, perm


# speedup vs baseline: 1.4198x; 1.4198x over previous
import jax
import jax.numpy as jnp
from jax import lax
from jax.experimental import pallas as pl
from jax.experimental.pallas import tpu as pltpu

_ALPHA = 0.4
_R = 64
_NBUF = 3
_INTERPRET = False


def _mix_body(perm_ref, ap_ref, aq_ref, lam_ref, hp_ref, hq_ref,
              op_ref, oq_ref, bufs_p, bufs_q, semp, semq):
    i = pl.program_id(0)
    n = pl.num_programs(0)
    rem = lax.rem(i, _NBUF)

    def issue(step, s):
        base = step * _R
        for j in range(_R):
            row = perm_ref[base + j]
            pltpu.make_async_copy(
                hp_ref.at[pl.ds(row, 1)], bufs_p[s].at[pl.ds(j, 1)],
                semp.at[s, j % 4]).start()
            pltpu.make_async_copy(
                hq_ref.at[pl.ds(row, 1)], bufs_q[s].at[pl.ds(j, 1)],
                semq.at[s, j % 4]).start()

    @pl.when(i == 0)
    def _():
        for k in range(_NBUF - 1):
            issue(k, k)

    lam = lam_ref[...]
    one_m = 1.0 - lam

    def step_for(s):
        s2 = (s + _NBUF - 1) % _NBUF

        @pl.when(i + _NBUF - 1 < n)
        def _():
            issue(i + _NBUF - 1, s2)

        for j in range(_R):
            pltpu.make_async_copy(
                hp_ref.at[pl.ds(0, 1)], bufs_p[s].at[pl.ds(j, 1)],
                semp.at[s, j % 4]).wait()
            pltpu.make_async_copy(
                hq_ref.at[pl.ds(0, 1)], bufs_q[s].at[pl.ds(j, 1)],
                semq.at[s, j % 4]).wait()
        a = ap_ref[...]
        b = bufs_p[s][...]
        op_ref[...] = a + jnp.log(lam + one_m * jnp.exp(b - a))
        a = aq_ref[...]
        b = bufs_q[s][...]
        oq_ref[...] = a + jnp.log(lam + one_m * jnp.exp(b - a))

    for s in range(_NBUF):
        @pl.when(rem == s)
        def _(s=s):
            step_for(s)


def kernel(x_pre, x_post):
    b, g = x_pre.shape
    key = jax.random.key(1)
    kp, kl = jax.random.split(key)
    perm = jax.random.permutation(kp, b)
    lam = jax.random.beta(kl, _ALPHA, _ALPHA, (b,)).astype(jnp.float32)

    hbm = pl.BlockSpec(memory_space=pl.ANY)
    lam_spec = pl.BlockSpec((_R, 1), lambda i, perm_r: (i, 0))
    out_spec = pl.BlockSpec((_R, g), lambda i, perm_r: (i, 0))

    grid_spec = pltpu.PrefetchScalarGridSpec(
        num_scalar_prefetch=1,
        grid=(b // _R,),
        in_specs=[out_spec, out_spec, lam_spec, hbm, hbm],
        out_specs=[out_spec, out_spec],
        scratch_shapes=[
            [pltpu.VMEM((_R, g), jnp.float32) for _ in range(_NBUF)],
            [pltpu.VMEM((_R, g), jnp.float32) for _ in range(_NBUF)],
            pltpu.SemaphoreType.DMA((_NBUF, 4)),
            pltpu.SemaphoreType.DMA((_NBUF, 4)),
        ],
    )
    out_shape = [jax.ShapeDtypeStruct((b, g), jnp.float32)] * 2
    op, oq = pl.pallas_call(
        _mix_body,
        grid_spec=grid_spec,
        out_shape=out_shape,
        interpret=_INTERPRET,
    )(perm, x_pre, x_post, lam.reshape(b, 1), x_pre, x_post)
    return op, oq, lam, perm
